# Initial kernel scaffold; baseline (speedup 1.0000x reference)
import jax
import jax.numpy as jnp
from jax import lax
from jax.experimental import pallas as pl
from jax.experimental.pallas import tpu as pltpu
from jax.experimental.pallas import tpu_sc as plsc

B = 16384
F = 26
D = 16
NW = 32
E = B // NW
C = 64
NCH = E // C
RPC = F * C
IW = 128
NG = RPC // IW


def _fm_body(x_r, emb, fc, wv, bv, out, idx2, rows2, fcv2, outv, pv, sems):
    wid = lax.axis_index("c") * 16 + lax.axis_index("s")

    pltpu.sync_copy(wv, pv.at[0])
    pltpu.sync_copy(bv, pv.at[1])

    def fire(g, b):
        row0 = (wid * NCH + g) * NG
        pltpu.sync_copy(x_r.at[pl.ds(row0, NG)], idx2.at[b])
        for r in range(NG):
            pltpu.async_copy(emb.at[idx2.at[b, r]],
                             rows2.at[b, pl.ds(r * IW, IW)], sems.at[b])
            pltpu.async_copy(fc.at[idx2.at[b, r]],
                             fcv2.at[b, pl.ds(r * IW, IW)], sems.at[b])

    def drain(b):
        for r in range(NG):
            pltpu.make_async_copy(emb.at[idx2.at[b, r]],
                                  rows2.at[b, pl.ds(r * IW, IW)],
                                  sems.at[b]).wait()
            pltpu.make_async_copy(fc.at[idx2.at[b, r]],
                                  fcv2.at[b, pl.ds(r * IW, IW)],
                                  sems.at[b]).wait()

    lane = lax.iota(jnp.int32, 16)

    def compute(g, b):
        wvec = pv[0, :]
        bvec = pv[1, :]
        for grp in range(C // 16):
            def elem(i, acc):
                e = grp * 16 + i
                s = jnp.zeros((16,), jnp.float32)
                q = jnp.zeros((16,), jnp.float32)
                for f in range(F):
                    v = rows2[b, f * C + e, :]
                    s = s + v
                    q = q + v * v
                r = jnp.sum(s * s - q)
                return jnp.where(lane == i, r, acc)

            acc = lax.fori_loop(0, 16, elem, jnp.zeros((16,), jnp.float32))
            fsum = jnp.zeros((16,), jnp.float32)
            for f in range(F):
                fsum = fsum + fcv2[b, pl.ds(f * C + grp * 16, 16)]
            tot = 0.5 * acc + wvec * fsum + bvec
            outv[pl.ds(grp * 16, 16)] = 1.0 / (1.0 + jnp.exp(-tot))
        pltpu.sync_copy(outv, out.at[pl.ds(wid * E + g * C, C)])

    fire(0, 0)

    @pl.loop(0, NCH, step=2)
    def _chunks(gg):
        fire(gg + 1, 1)
        drain(0)
        compute(gg, 0)

        @pl.when(gg + 2 < NCH)
        def _refill():
            fire(gg + 2, 0)

        drain(1)
        compute(gg + 1, 1)


def kernel(x, emb_table, fc_table, lin_w, lin_b):
    x_r = (x.astype(jnp.int32)
             .reshape(NW, NCH, C, F)
             .transpose(0, 1, 3, 2)
             .reshape(NW * NCH * NG, IW))
    fc_flat = fc_table.reshape(-1)
    wv = jnp.broadcast_to(lin_w.reshape(()), (16,)).astype(jnp.float32)
    bv = jnp.broadcast_to(lin_b.reshape(()), (16,)).astype(jnp.float32)

    mesh = plsc.VectorSubcoreMesh(core_axis_name="c", subcore_axis_name="s")
    out = pl.kernel(
        _fm_body,
        out_type=jax.ShapeDtypeStruct((B,), jnp.float32),
        mesh=mesh,
        scratch_types=[
            pltpu.VMEM((2, NG, IW), jnp.int32),
            pltpu.VMEM((2, RPC, D), jnp.float32),
            pltpu.VMEM((2, RPC), jnp.float32),
            pltpu.VMEM((C,), jnp.float32),
            pltpu.VMEM((2, 16), jnp.float32),
            pltpu.SemaphoreType.DMA((2,)),
        ],
    )(x_r, emb_table, fc_flat, wv, bv)
    return out.reshape(B, 1)


# speedup vs baseline: 1.3783x; 1.3783x over previous
import jax
import jax.numpy as jnp
from jax import lax
from jax.experimental import pallas as pl
from jax.experimental.pallas import tpu as pltpu
from jax.experimental.pallas import tpu_sc as plsc

B = 16384
F = 26
D = 16
NW = 32
E = B // NW
C = 64
NCH = E // C
RPC = F * C
IW = 128
NG = RPC // IW
NGP = 16


def _fm_body(x_r, emb, fc, wv, bv, out, idx2, rows2, fcv2, outv, tbuf, pv, sems):
    wid = lax.axis_index("c") * 16 + lax.axis_index("s")

    pltpu.sync_copy(wv, pv.at[0])
    pltpu.sync_copy(bv, pv.at[1])

    def fire(g, b):
        row0 = (wid * NCH + g) * NGP
        pltpu.sync_copy(x_r.at[pl.ds(row0, NGP)], idx2.at[b])
        for r in range(NG):
            pltpu.async_copy(emb.at[idx2.at[b, r]],
                             rows2.at[b, pl.ds(r * IW, IW)], sems.at[b])
            pltpu.async_copy(fc.at[idx2.at[b, r]],
                             fcv2.at[b, pl.ds(r * IW, IW)], sems.at[b])

    def drain(b):
        for r in range(NG):
            pltpu.make_async_copy(emb.at[idx2.at[b, r]],
                                  rows2.at[b, pl.ds(r * IW, IW)],
                                  sems.at[b]).wait()
            pltpu.make_async_copy(fc.at[idx2.at[b, r]],
                                  fcv2.at[b, pl.ds(r * IW, IW)],
                                  sems.at[b]).wait()

    lane16 = lax.iota(jnp.int32, 16) * 16

    def compute(g, b):
        wvec = pv[0, :]
        bvec = pv[1, :]
        for grp in range(C // 16):
            @pl.loop(0, 16)
            def _elem(i):
                e = grp * 16 + i
                s = jnp.zeros((16,), jnp.float32)
                q = jnp.zeros((16,), jnp.float32)
                for f in range(F):
                    v = rows2[b, f * C + e, :]
                    s = s + v
                    q = q + v * v
                tbuf[pl.ds(i * 16, 16)] = s * s - q

            acc = jnp.zeros((16,), jnp.float32)
            for dcol in range(16):
                acc = acc + plsc.load_gather(tbuf, [lane16 + dcol])
            fsum = jnp.zeros((16,), jnp.float32)
            for f in range(F):
                fsum = fsum + fcv2[b, pl.ds(f * C + grp * 16, 16)]
            tot = 0.5 * acc + wvec * fsum + bvec
            outv[pl.ds(grp * 16, 16)] = 1.0 / (1.0 + jnp.exp(-tot))
        pltpu.sync_copy(outv, out.at[pl.ds(wid * E + g * C, C)])

    fire(0, 0)

    @pl.loop(0, NCH, step=2)
    def _chunks(gg):
        fire(gg + 1, 1)
        drain(0)
        compute(gg, 0)

        @pl.when(gg + 2 < NCH)
        def _refill():
            fire(gg + 2, 0)

        drain(1)
        compute(gg + 1, 1)


def kernel(x, emb_table, fc_table, lin_w, lin_b):
    x_r = (x.astype(jnp.int32)
             .reshape(NW, NCH, C, F)
             .transpose(0, 1, 3, 2)
             .reshape(NW, NCH, NG, IW))
    x_r = jnp.pad(x_r, ((0, 0), (0, 0), (0, NGP - NG), (0, 0)))
    x_r = x_r.reshape(NW * NCH * NGP, IW)
    fc_flat = fc_table.reshape(-1)
    wv = jnp.broadcast_to(lin_w.reshape(()), (16,)).astype(jnp.float32)
    bv = jnp.broadcast_to(lin_b.reshape(()), (16,)).astype(jnp.float32)

    mesh = plsc.VectorSubcoreMesh(core_axis_name="c", subcore_axis_name="s")
    out = pl.kernel(
        _fm_body,
        out_type=jax.ShapeDtypeStruct((B,), jnp.float32),
        mesh=mesh,
        compiler_params=pltpu.CompilerParams(needs_layout_passes=False,
                                             use_tc_tiling_on_sc=False),
        scratch_types=[
            pltpu.VMEM((2, NGP, IW), jnp.int32),
            pltpu.VMEM((2, RPC, D), jnp.float32),
            pltpu.VMEM((2, RPC), jnp.float32),
            pltpu.VMEM((C,), jnp.float32),
            pltpu.VMEM((256,), jnp.float32),
            pltpu.VMEM((2, 16), jnp.float32),
            pltpu.SemaphoreType.DMA((2,)),
        ],
    )(x_r, emb_table, fc_flat, wv, bv)
    return out.reshape(B, 1)
